# baseline (speedup 1.0000x reference)
import functools

import jax
import jax.numpy as jnp
from jax import lax
from jax.experimental import pallas as pl
from jax.experimental.pallas import tpu as pltpu
from jax.experimental.pallas import tpu_sc as plsc

_NC = 2
_NS = 16
_NW = _NC * _NS

_ROWS_PER_CHUNK = 16
_XBUFS = 4
_PBUFS = 2


def kernel(x, pe_table):
    B, L, D = x.shape
    chunk = _ROWS_PER_CHUNK * D
    rows_w = L // _NW
    n_chunks = rows_w // _ROWS_PER_CHUNK
    n_iters = n_chunks * B
    pe_words_w = rows_w * D
    x_words_b = L * D

    mesh = plsc.VectorSubcoreMesh(
        core_axis_name="c", subcore_axis_name="s",
        num_cores=_NC, num_subcores=_NS)

    @functools.partial(
        pl.kernel,
        out_type=jax.ShapeDtypeStruct((B * L * D,), jnp.float32),
        mesh=mesh,
        scratch_types=(
            [pltpu.VMEM((chunk,), jnp.float32) for _ in range(_XBUFS)]
            + [pltpu.VMEM((chunk,), jnp.float32) for _ in range(_PBUFS)]
            + [pltpu.SemaphoreType.DMA for _ in range(_XBUFS)]
            + [pltpu.SemaphoreType.DMA for _ in range(_XBUFS)]
            + [pltpu.SemaphoreType.DMA for _ in range(_PBUFS)]
        ),
    )
    def run(x_hbm, pe_hbm, out_hbm, *scratch):
        bufx = scratch[:_XBUFS]
        bufp = scratch[_XBUFS:_XBUFS + _PBUFS]
        ld_sem = scratch[_XBUFS + _PBUFS:2 * _XBUFS + _PBUFS]
        st_sem = scratch[2 * _XBUFS + _PBUFS:3 * _XBUFS + _PBUFS]
        pe_sem = scratch[3 * _XBUFS + _PBUFS:]

        wid = lax.axis_index("s") * _NC + lax.axis_index("c")
        pe_base = wid * pe_words_w

        def x_off(i):
            c, b = i // B, i % B
            return b * x_words_b + pe_base + c * chunk

        def pe_load(c, sp):
            return pltpu.make_async_copy(
                pe_hbm.at[pl.ds(pe_base + c * chunk, chunk)],
                bufp[sp], pe_sem[sp])

        def x_load(i, s):
            return pltpu.make_async_copy(
                x_hbm.at[pl.ds(x_off(i), chunk)], bufx[s], ld_sem[s])

        def out_store(i, s):
            return pltpu.make_async_copy(
                bufx[s], out_hbm.at[pl.ds(x_off(i), chunk)], st_sem[s])

        pe_load(0, 0).start()
        for k in range(_XBUFS - 1):
            x_load(jnp.int32(k), k).start()

        def body(j, carry):
            for half in range(2):
                sp = half
                c = 2 * j + half
                pe_load(c, sp).wait()

                @pl.when(c + 1 < n_chunks)
                def _():
                    pe_load(c + 1, 1 - sp).start()

                for b in range(B):
                    ks = (B * half + b) % _XBUFS
                    i = B * (2 * j + half) + b
                    s = ks
                    sprev = (ks - 1) % _XBUFS

                    @pl.when(jnp.logical_and(i - 1 >= 0,
                                             i + _XBUFS - 1 < n_iters))
                    def _():
                        out_store(i - 1, sprev).wait()

                    @pl.when(i + _XBUFS - 1 < n_iters)
                    def _():
                        x_load(i + _XBUFS - 1, sprev).start()

                    x_load(i, s).wait()

                    bx, bp = bufx[s], bufp[sp]

                    @plsc.parallel_loop(0, chunk, step=16, unroll=8)
                    def _(o):
                        plsc.addupdate(bx.at[pl.ds(o, 16)],
                                       bp[pl.ds(o, 16)])

                    out_store(i, s).start()
            return carry

        lax.fori_loop(0, n_chunks // 2, body, 0)

        for k in range(n_iters - _XBUFS, n_iters):
            out_store(jnp.int32(k), k % _XBUFS).wait()

    out = run(x.reshape(-1), pe_table[:L].reshape(-1))
    return out.reshape(B, L, D)


# speedup vs baseline: 2.6454x; 2.6454x over previous
import functools

import jax
import jax.numpy as jnp
from jax import lax
from jax.experimental import pallas as pl
from jax.experimental.pallas import tpu as pltpu
from jax.experimental.pallas import tpu_sc as plsc

_NC = 2
_NS = 16
_NW = _NC * _NS

_ROWS_PER_CHUNK = 16
_XBUFS = 4
_PBUFS = 2


def kernel(x, pe_table):
    B, L, D = x.shape
    rows_w = L // _NW
    n_chunks = rows_w // _ROWS_PER_CHUNK
    n_iters = n_chunks * B

    mesh = plsc.VectorSubcoreMesh(
        core_axis_name="c", subcore_axis_name="s",
        num_cores=_NC, num_subcores=_NS)

    @functools.partial(
        pl.kernel,
        out_type=jax.ShapeDtypeStruct((B * L, D), jnp.float32),
        mesh=mesh,
        scratch_types=(
            [pltpu.VMEM((_ROWS_PER_CHUNK, D), jnp.float32)
             for _ in range(_XBUFS + _PBUFS)]
            + [pltpu.SemaphoreType.DMA for _ in range(_XBUFS)]
            + [pltpu.SemaphoreType.DMA for _ in range(_XBUFS)]
            + [pltpu.SemaphoreType.DMA for _ in range(_PBUFS)]
        ),
    )
    def run(x_hbm, pe_hbm, out_hbm, *scratch):
        bufx = scratch[:_XBUFS]
        bufp = scratch[_XBUFS:_XBUFS + _PBUFS]
        ld_sem = scratch[_XBUFS + _PBUFS:2 * _XBUFS + _PBUFS]
        st_sem = scratch[2 * _XBUFS + _PBUFS:3 * _XBUFS + _PBUFS]
        pe_sem = scratch[3 * _XBUFS + _PBUFS:]

        wid = lax.axis_index("s") * _NC + lax.axis_index("c")
        row_base = wid * rows_w

        def x_row(i):
            c, b = i // B, i % B
            return b * L + row_base + c * _ROWS_PER_CHUNK

        def pe_load(c, sp):
            return pltpu.make_async_copy(
                pe_hbm.at[pl.ds(row_base + c * _ROWS_PER_CHUNK,
                                _ROWS_PER_CHUNK)],
                bufp[sp], pe_sem[sp])

        def x_load(i, s):
            return pltpu.make_async_copy(
                x_hbm.at[pl.ds(x_row(i), _ROWS_PER_CHUNK)],
                bufx[s], ld_sem[s])

        def out_store(i, s):
            return pltpu.make_async_copy(
                bufx[s], out_hbm.at[pl.ds(x_row(i), _ROWS_PER_CHUNK)],
                st_sem[s])

        pe_load(0, 0).start()
        for k in range(_XBUFS - 1):
            x_load(jnp.int32(k), k).start()

        def body(j, carry):
            for half in range(2):
                sp = half
                c = 2 * j + half
                pe_load(c, sp).wait()

                @pl.when(c + 1 < n_chunks)
                def _():
                    pe_load(c + 1, 1 - sp).start()

                for b in range(B):
                    ks = (B * half + b) % _XBUFS
                    i = B * (2 * j + half) + b
                    s = ks
                    sprev = (ks - 1) % _XBUFS

                    @pl.when(jnp.logical_and(i - 1 >= 0,
                                             i + _XBUFS - 1 < n_iters))
                    def _():
                        out_store(i - 1, sprev).wait()

                    @pl.when(i + _XBUFS - 1 < n_iters)
                    def _():
                        x_load(i + _XBUFS - 1, sprev).start()

                    x_load(i, s).wait()

                    bx, bp = bufx[s], bufp[sp]

                    for r in range(_ROWS_PER_CHUNK):
                        @plsc.parallel_loop(0, D, step=16, unroll=8)
                        def _(o):
                            plsc.addupdate(bx.at[r, pl.ds(o, 16)],
                                           bp[r, pl.ds(o, 16)])

                    out_store(i, s).start()
            return carry

        lax.fori_loop(0, n_chunks // 2, body, 0)

        for k in range(n_iters - _XBUFS, n_iters):
            out_store(jnp.int32(k), k % _XBUFS).wait()

    out = run(x.reshape(B * L, D), pe_table[:L])
    return out.reshape(B, L, D)


# speedup vs baseline: 2.8723x; 1.0858x over previous
import functools

import jax
import jax.numpy as jnp
from jax import lax
from jax.experimental import pallas as pl
from jax.experimental.pallas import tpu as pltpu
from jax.experimental.pallas import tpu_sc as plsc

_NC = 2
_NS = 16
_NW = _NC * _NS

_ROWS_PER_CHUNK = 16
_XBUFS = 4
_PBUFS = 2


def kernel(x, pe_table):
    B, L, D = x.shape
    rows_w = L // _NW
    n_chunks = rows_w // _ROWS_PER_CHUNK
    n_iters = n_chunks * B

    mesh = plsc.VectorSubcoreMesh(
        core_axis_name="c", subcore_axis_name="s",
        num_cores=_NC, num_subcores=_NS)

    @functools.partial(
        pl.kernel,
        out_type=jax.ShapeDtypeStruct((B * L, D), jnp.float32),
        mesh=mesh,
        scratch_types=(
            [pltpu.VMEM((_ROWS_PER_CHUNK, D), jnp.float32)
             for _ in range(_XBUFS + _PBUFS)]
            + [pltpu.SemaphoreType.DMA for _ in range(_XBUFS)]
            + [pltpu.SemaphoreType.DMA for _ in range(_XBUFS)]
            + [pltpu.SemaphoreType.DMA for _ in range(_PBUFS)]
        ),
    )
    def run(x_hbm, pe_hbm, out_hbm, *scratch):
        bufx = scratch[:_XBUFS]
        bufp = scratch[_XBUFS:_XBUFS + _PBUFS]
        ld_sem = scratch[_XBUFS + _PBUFS:2 * _XBUFS + _PBUFS]
        st_sem = scratch[2 * _XBUFS + _PBUFS:3 * _XBUFS + _PBUFS]
        pe_sem = scratch[3 * _XBUFS + _PBUFS:]

        wid = lax.axis_index("s") * _NC + lax.axis_index("c")
        row_base = wid * rows_w

        def x_row(i):
            c, b = i // B, i % B
            return b * L + row_base + c * _ROWS_PER_CHUNK

        def pe_load(c, sp):
            return pltpu.make_async_copy(
                pe_hbm.at[pl.ds(row_base + c * _ROWS_PER_CHUNK,
                                _ROWS_PER_CHUNK)],
                bufp[sp], pe_sem[sp])

        def x_load(i, s):
            return pltpu.make_async_copy(
                x_hbm.at[pl.ds(x_row(i), _ROWS_PER_CHUNK)],
                bufx[s], ld_sem[s])

        def out_store(i, s):
            return pltpu.make_async_copy(
                bufx[s], out_hbm.at[pl.ds(x_row(i), _ROWS_PER_CHUNK)],
                st_sem[s])

        pe_load(0, 0).start()
        for k in range(_XBUFS - 1):
            x_load(jnp.int32(k), k).start()

        def body(j, carry):
            for half in range(2):
                sp = half
                c = 2 * j + half
                pe_load(c, sp).wait()

                @pl.when(c + 1 < n_chunks)
                def _():
                    pe_load(c + 1, 1 - sp).start()

                for b in range(B):
                    ks = (B * half + b) % _XBUFS
                    i = B * (2 * j + half) + b
                    s = ks
                    sprev = (ks - 1) % _XBUFS

                    @pl.when(jnp.logical_and(i - 1 >= 0,
                                             i + _XBUFS - 1 < n_iters))
                    def _():
                        out_store(i - 1, sprev).wait()

                    @pl.when(i + _XBUFS - 1 < n_iters)
                    def _():
                        x_load(i + _XBUFS - 1, sprev).start()

                    x_load(i, s).wait()

                    bx, bp = bufx[s], bufp[sp]

                    @plsc.parallel_loop(0, _ROWS_PER_CHUNK * D,
                                        step=16, unroll=8)
                    def _(o):
                        r = o // D
                        cc = o % D
                        plsc.addupdate(bx.at[r, pl.ds(cc, 16)],
                                       bp[r, pl.ds(cc, 16)])

                    out_store(i, s).start()
            return carry

        lax.fori_loop(0, n_chunks // 2, body, 0)

        for k in range(n_iters - _XBUFS, n_iters):
            out_store(jnp.int32(k), k % _XBUFS).wait()

    out = run(x.reshape(B * L, D), pe_table[:L])
    return out.reshape(B, L, D)
